# Initial kernel scaffold; baseline (speedup 1.0000x reference)
import jax
import jax.numpy as jnp
from jax.experimental import pallas as pl


def kernel(x, edge_index, edge_attr, params):
    raise NotImplementedError("write your pallas kernel here")


# speedup vs baseline: 3.6337x; 3.6337x over previous
import functools

import jax
import jax.numpy as jnp
from jax import lax
from jax.experimental import pallas as pl
from jax.experimental.pallas import tpu as pltpu
from jax.experimental.pallas import tpu_sc as plsc

_H = 128
_NC = 2
_NS = 16
_NW = _NC * _NS
_CH = 80
_ZR = 128

_BN = 2000
_BE = 4000


def _ln(h, g, b):
    mu = jnp.mean(h, axis=-1, keepdims=True)
    d = h - mu
    var = jnp.mean(d * d, axis=-1, keepdims=True)
    return g * d * lax.rsqrt(var + 1e-5) + b


def _row2(block):
    return pl.BlockSpec(block, lambda i: (i, 0))


def _fix2(block):
    return pl.BlockSpec(block, lambda i: (0, 0))


def _node_enc(x, p, wa_next, wb_next):
    n, din = x.shape

    def body(x_r, w1_r, b1_r, w2_r, b2_r, g_r, bb_r, wa_r, wb_r, h_r, a_r, b_r):
        h = jnp.maximum(x_r[...] @ w1_r[...] + b1_r[...], 0.0)
        h = _ln(h @ w2_r[...] + b2_r[...], g_r[...], bb_r[...])
        h_r[...] = h
        a_r[...] = h @ wa_r[...]
        b_r[...] = h @ wb_r[...]

    out = pl.pallas_call(
        body,
        grid=(n // _BN,),
        in_specs=[
            _row2((_BN, din)),
            _fix2((din, _H)), _fix2((1, _H)),
            _fix2((_H, _H)), _fix2((1, _H)),
            _fix2((1, _H)), _fix2((1, _H)),
            _fix2((_H, _H)), _fix2((_H, _H)),
        ],
        out_specs=[_row2((_BN, _H))] * 3,
        out_shape=[jax.ShapeDtypeStruct((n, _H), jnp.float32)] * 3,
    )(x, p["w1"], p["b1"].reshape(1, _H), p["w2"], p["b2"].reshape(1, _H),
      p["ln_g"].reshape(1, _H), p["ln_b"].reshape(1, _H), wa_next, wb_next)
    return out


def _edge_enc(ea, p):
    n, din = ea.shape

    def body(x_r, w1_r, b1_r, w2_r, b2_r, g_r, bb_r, o_r):
        h = jnp.maximum(x_r[...] @ w1_r[...] + b1_r[...], 0.0)
        o_r[...] = _ln(h @ w2_r[...] + b2_r[...], g_r[...], bb_r[...])

    return pl.pallas_call(
        body,
        grid=(n // _BE,),
        in_specs=[
            _row2((_BE, din)),
            _fix2((din, _H)), _fix2((1, _H)),
            _fix2((_H, _H)), _fix2((1, _H)),
            _fix2((1, _H)), _fix2((1, _H)),
        ],
        out_specs=_row2((_BE, _H)),
        out_shape=jax.ShapeDtypeStruct((n, _H), jnp.float32),
    )(ea, p["w1"], p["b1"].reshape(1, _H), p["w2"], p["b2"].reshape(1, _H),
      p["ln_g"].reshape(1, _H), p["ln_b"].reshape(1, _H))


def _edge_mlp(gsum, e, w1c, p):
    n = gsum.shape[0]

    def body(g_r, e_r, w1c_r, b1_r, w2_r, b2_r, lg_r, lb_r, o_r):
        ev = e_r[...]
        hid = jnp.maximum(g_r[...] + ev @ w1c_r[...] + b1_r[...], 0.0)
        o_r[...] = _ln(hid @ w2_r[...] + b2_r[...], lg_r[...], lb_r[...]) + ev

    return pl.pallas_call(
        body,
        grid=(n // _BE,),
        in_specs=[
            _row2((_BE, _H)), _row2((_BE, _H)),
            _fix2((_H, _H)), _fix2((1, _H)),
            _fix2((_H, _H)), _fix2((1, _H)),
            _fix2((1, _H)), _fix2((1, _H)),
        ],
        out_specs=_row2((_BE, _H)),
        out_shape=jax.ShapeDtypeStruct((n, _H), jnp.float32),
    )(gsum, e, w1c, p["b1"].reshape(1, _H), p["w2"], p["b2"].reshape(1, _H),
      p["ln_g"].reshape(1, _H), p["ln_b"].reshape(1, _H))


def _node_mlp(h, p0, p1, p, wa_next=None, wb_next=None):
    n = h.shape[0]
    wa = p["w1"][:_H]
    wb = p["w1"][_H:]
    proj = wa_next is not None

    def body(h_r, p0_r, p1_r, wa_r, wb_r, b1_r, w2_r, b2_r, lg_r, lb_r, *rest):
        hv = h_r[...]
        agg = p0_r[...] + p1_r[...]
        hid = jnp.maximum(hv @ wa_r[...] + agg @ wb_r[...] + b1_r[...], 0.0)
        hn = _ln(hid @ w2_r[...] + b2_r[...], lg_r[...], lb_r[...]) + hv
        if proj:
            wan_r, wbn_r, h_o, a_o, b_o = rest
            a_o[...] = hn @ wan_r[...]
            b_o[...] = hn @ wbn_r[...]
        else:
            (h_o,) = rest
        h_o[...] = hn

    in_specs = [
        _row2((_BN, _H)), _row2((_BN, _H)), _row2((_BN, _H)),
        _fix2((_H, _H)), _fix2((_H, _H)), _fix2((1, _H)),
        _fix2((_H, _H)), _fix2((1, _H)),
        _fix2((1, _H)), _fix2((1, _H)),
    ]
    args = [h, p0, p1, wa, wb, p["b1"].reshape(1, _H), p["w2"],
            p["b2"].reshape(1, _H), p["ln_g"].reshape(1, _H),
            p["ln_b"].reshape(1, _H)]
    n_out = 3 if proj else 1
    if proj:
        in_specs += [_fix2((_H, _H)), _fix2((_H, _H))]
        args += [wa_next, wb_next]
    out = pl.pallas_call(
        body,
        grid=(n // _BN,),
        in_specs=in_specs,
        out_specs=[_row2((_BN, _H))] * n_out,
        out_shape=[jax.ShapeDtypeStruct((n, _H), jnp.float32)] * n_out,
    )(*args)
    return out if proj else out[0]


def _decoder(h, w1, b1, w2, b2):
    n = h.shape[0]
    dout = w2.shape[1]

    def body(h_r, w1_r, b1_r, w2_r, b2_r, o_r):
        hid = jnp.maximum(h_r[...] @ w1_r[...] + b1_r[...], 0.0)
        o_r[...] = hid @ w2_r[...] + b2_r[...]

    return pl.pallas_call(
        body,
        grid=(n // _BN,),
        in_specs=[
            _row2((_BN, _H)),
            _fix2((_H, _H)), _fix2((1, _H)),
            _fix2((_H, dout)), _fix2((1, dout)),
        ],
        out_specs=_row2((_BN, dout)),
        out_shape=jax.ShapeDtypeStruct((n, dout), jnp.float32),
    )(h, w1, b1.reshape(1, _H), w2, b2.reshape(1, dout))


def _sc_gather_sum(a_tab, b_tab, src, dst):
    e = src.shape[0]
    per_tile = e // _NW
    n_ch = per_tile // _CH
    assert per_tile * _NW == e and n_ch * _CH == per_tile

    mesh = plsc.VectorSubcoreMesh(core_axis_name="c", subcore_axis_name="s")

    @functools.partial(
        pl.kernel,
        mesh=mesh,
        out_type=jax.ShapeDtypeStruct((e, _H), jnp.float32),
        scratch_types=[
            pltpu.VMEM((_CH,), jnp.int32),
            pltpu.VMEM((_CH,), jnp.int32),
            pltpu.VMEM((_CH, _H), jnp.float32),
            pltpu.VMEM((_CH, _H), jnp.float32),
            pltpu.SemaphoreType.DMA,
            pltpu.SemaphoreType.DMA,
        ],
    )
    def k(a_h, b_h, src_h, dst_h, out_h, idxs, idxd, buf_a, buf_b, sem_a, sem_b):
        c = lax.axis_index("c")
        s = lax.axis_index("s")
        tbase = (c * _NS + s) * per_tile

        def chunk(i, _):
            base = pl.multiple_of(tbase + i * _CH, 8)
            pltpu.sync_copy(src_h.at[pl.ds(base, _CH)], idxs)
            pltpu.sync_copy(dst_h.at[pl.ds(base, _CH)], idxd)
            cp_a = pltpu.async_copy(a_h.at[idxs], buf_a, sem_a)
            cp_b = pltpu.async_copy(b_h.at[idxd], buf_b, sem_b)
            cp_a.wait()
            cp_b.wait()

            def add_row(r, _):
                for j in range(_H // 16):
                    sl = pl.ds(j * 16, 16)
                    buf_a[r, sl] = buf_a[r, sl] + buf_b[r, sl]
                return 0

            lax.fori_loop(0, _CH, add_row, 0, unroll=False)
            pltpu.sync_copy(buf_a, out_h.at[pl.ds(base, _CH)])
            return 0

        lax.fori_loop(0, n_ch, chunk, 0, unroll=False)

    return k(a_tab, b_tab, src, dst)


def _sc_scatter_sum(eu, dst, n_nodes):
    e = eu.shape[0]
    per_tile = e // _NW
    n_ch = per_tile // _CH
    rows_per_sub = -(-n_nodes // (_NS * _ZR)) * _ZR
    n_pad = rows_per_sub * _NS
    n_z = rows_per_sub // _ZR
    assert n_ch * _CH == per_tile and n_z * _ZR == rows_per_sub

    mesh = plsc.VectorSubcoreMesh(core_axis_name="c", subcore_axis_name="s")

    @functools.partial(
        pl.kernel,
        mesh=mesh,
        out_type=jax.ShapeDtypeStruct((_NC, n_pad, _H), jnp.float32),
        scratch_types=[
            pltpu.VMEM((_CH,), jnp.int32),
            pltpu.VMEM((_CH, _H), jnp.float32),
            pltpu.VMEM((_ZR, _H), jnp.float32),
            pltpu.VMEM_SHARED((n_pad, _H), jnp.float32),
        ],
    )
    def k(eu_h, dst_h, out_h, idxd, buf, zbuf, accum):
        c = lax.axis_index("c")
        s = lax.axis_index("s")
        tbase = (c * _NS + s) * per_tile

        def zrow(r, _):
            for j in range(_H // 16):
                zbuf[r, pl.ds(j * 16, 16)] = jnp.zeros((16,), jnp.float32)
            return 0

        lax.fori_loop(0, _ZR, zrow, 0, unroll=False)
        for z in range(n_z):
            pltpu.sync_copy(zbuf, accum.at[pl.ds(s * rows_per_sub + z * _ZR, _ZR)])
        plsc.subcore_barrier()

        def chunk(i, _):
            base = pl.multiple_of(tbase + i * _CH, 8)
            pltpu.sync_copy(dst_h.at[pl.ds(base, _CH)], idxd)
            pltpu.sync_copy(eu_h.at[pl.ds(base, _CH)], buf)
            pltpu.sync_copy(buf, accum.at[idxd], add=True)
            return 0

        lax.fori_loop(0, n_ch, chunk, 0, unroll=False)
        plsc.subcore_barrier()
        pltpu.sync_copy(accum.at[pl.ds(s * rows_per_sub, rows_per_sub)],
                        out_h.at[c, pl.ds(s * rows_per_sub, rows_per_sub)])

    return k(eu, dst)[:, :n_nodes]


def kernel(x, edge_index, edge_attr, params):
    n_nodes = x.shape[0]
    src = edge_index[0]
    dst = edge_index[1]

    pe0 = params["proc0_edge"]
    pn0 = params["proc0_node"]
    pe1 = params["proc1_edge"]
    pn1 = params["proc1_node"]

    h, a0, b0 = _node_enc(x, params["node_enc"], pe0["w1"][:_H], pe0["w1"][_H:2 * _H])
    e = _edge_enc(edge_attr, params["edge_enc"])

    g0 = _sc_gather_sum(a0, b0, src, dst)
    eu0 = _edge_mlp(g0, e, pe0["w1"][2 * _H:], pe0)
    part0 = _sc_scatter_sum(eu0, dst, n_nodes)
    h, a1, b1 = _node_mlp(h, part0[0], part0[1], pn0,
                          pe1["w1"][:_H], pe1["w1"][_H:2 * _H])

    g1 = _sc_gather_sum(a1, b1, src, dst)
    eu1 = _edge_mlp(g1, eu0, pe1["w1"][2 * _H:], pe1)
    part1 = _sc_scatter_sum(eu1, dst, n_nodes)
    h = _node_mlp(h, part1[0], part1[1], pn1)

    return _decoder(h, params["dec_w1"], params["dec_b1"],
                    params["dec_w2"], params["dec_b2"])


# speedup vs baseline: 5.2750x; 1.4517x over previous
import functools

import jax
import jax.numpy as jnp
from jax import lax
from jax.experimental import pallas as pl
from jax.experimental.pallas import tpu as pltpu
from jax.experimental.pallas import tpu_sc as plsc

_H = 128
_NC = 2
_NS = 16
_NW = _NC * _NS
_CH = 80
_ZR = 128

_BN = 2000
_BE = 4000


def _ln(h, g, b):
    mu = jnp.mean(h, axis=-1, keepdims=True)
    d = h - mu
    var = jnp.mean(d * d, axis=-1, keepdims=True)
    return g * d * lax.rsqrt(var + 1e-5) + b


def _row2(block):
    return pl.BlockSpec(block, lambda i: (i, 0))


def _fix2(block):
    return pl.BlockSpec(block, lambda i: (0, 0))


def _node_enc(x, p, wa_next, wb_next):
    n, din = x.shape

    def body(x_r, w1_r, b1_r, w2_r, b2_r, g_r, bb_r, wa_r, wb_r, h_r, a_r, b_r):
        h = jnp.maximum(x_r[...] @ w1_r[...] + b1_r[...], 0.0)
        h = _ln(h @ w2_r[...] + b2_r[...], g_r[...], bb_r[...])
        h_r[...] = h
        a_r[...] = h @ wa_r[...]
        b_r[...] = h @ wb_r[...]

    out = pl.pallas_call(
        body,
        grid=(n // _BN,),
        in_specs=[
            _row2((_BN, din)),
            _fix2((din, _H)), _fix2((1, _H)),
            _fix2((_H, _H)), _fix2((1, _H)),
            _fix2((1, _H)), _fix2((1, _H)),
            _fix2((_H, _H)), _fix2((_H, _H)),
        ],
        out_specs=[_row2((_BN, _H))] * 3,
        out_shape=[jax.ShapeDtypeStruct((n, _H), jnp.float32)] * 3,
    )(x, p["w1"], p["b1"].reshape(1, _H), p["w2"], p["b2"].reshape(1, _H),
      p["ln_g"].reshape(1, _H), p["ln_b"].reshape(1, _H), wa_next, wb_next)
    return out


def _edge_enc(ea, p):
    n, din = ea.shape

    def body(x_r, w1_r, b1_r, w2_r, b2_r, g_r, bb_r, o_r):
        h = jnp.maximum(x_r[...] @ w1_r[...] + b1_r[...], 0.0)
        o_r[...] = _ln(h @ w2_r[...] + b2_r[...], g_r[...], bb_r[...])

    return pl.pallas_call(
        body,
        grid=(n // _BE,),
        in_specs=[
            _row2((_BE, din)),
            _fix2((din, _H)), _fix2((1, _H)),
            _fix2((_H, _H)), _fix2((1, _H)),
            _fix2((1, _H)), _fix2((1, _H)),
        ],
        out_specs=_row2((_BE, _H)),
        out_shape=jax.ShapeDtypeStruct((n, _H), jnp.float32),
    )(ea, p["w1"], p["b1"].reshape(1, _H), p["w2"], p["b2"].reshape(1, _H),
      p["ln_g"].reshape(1, _H), p["ln_b"].reshape(1, _H))


def _edge_mlp(gsum, e, w1c, p):
    n = gsum.shape[0]

    def body(g_r, e_r, w1c_r, b1_r, w2_r, b2_r, lg_r, lb_r, o_r):
        ev = e_r[...]
        hid = jnp.maximum(g_r[...] + ev @ w1c_r[...] + b1_r[...], 0.0)
        o_r[...] = _ln(hid @ w2_r[...] + b2_r[...], lg_r[...], lb_r[...]) + ev

    return pl.pallas_call(
        body,
        grid=(n // _BE,),
        in_specs=[
            _row2((_BE, _H)), _row2((_BE, _H)),
            _fix2((_H, _H)), _fix2((1, _H)),
            _fix2((_H, _H)), _fix2((1, _H)),
            _fix2((1, _H)), _fix2((1, _H)),
        ],
        out_specs=_row2((_BE, _H)),
        out_shape=jax.ShapeDtypeStruct((n, _H), jnp.float32),
    )(gsum, e, w1c, p["b1"].reshape(1, _H), p["w2"], p["b2"].reshape(1, _H),
      p["ln_g"].reshape(1, _H), p["ln_b"].reshape(1, _H))


def _node_mlp(h, p0, p1, p, wa_next=None, wb_next=None):
    n = h.shape[0]
    wa = p["w1"][:_H]
    wb = p["w1"][_H:]
    proj = wa_next is not None

    def body(h_r, p0_r, p1_r, wa_r, wb_r, b1_r, w2_r, b2_r, lg_r, lb_r, *rest):
        hv = h_r[...]
        agg = p0_r[...] + p1_r[...]
        hid = jnp.maximum(hv @ wa_r[...] + agg @ wb_r[...] + b1_r[...], 0.0)
        hn = _ln(hid @ w2_r[...] + b2_r[...], lg_r[...], lb_r[...]) + hv
        if proj:
            wan_r, wbn_r, h_o, a_o, b_o = rest
            a_o[...] = hn @ wan_r[...]
            b_o[...] = hn @ wbn_r[...]
        else:
            (h_o,) = rest
        h_o[...] = hn

    in_specs = [
        _row2((_BN, _H)), _row2((_BN, _H)), _row2((_BN, _H)),
        _fix2((_H, _H)), _fix2((_H, _H)), _fix2((1, _H)),
        _fix2((_H, _H)), _fix2((1, _H)),
        _fix2((1, _H)), _fix2((1, _H)),
    ]
    args = [h, p0, p1, wa, wb, p["b1"].reshape(1, _H), p["w2"],
            p["b2"].reshape(1, _H), p["ln_g"].reshape(1, _H),
            p["ln_b"].reshape(1, _H)]
    n_out = 3 if proj else 1
    if proj:
        in_specs += [_fix2((_H, _H)), _fix2((_H, _H))]
        args += [wa_next, wb_next]
    out = pl.pallas_call(
        body,
        grid=(n // _BN,),
        in_specs=in_specs,
        out_specs=[_row2((_BN, _H))] * n_out,
        out_shape=[jax.ShapeDtypeStruct((n, _H), jnp.float32)] * n_out,
    )(*args)
    return out if proj else out[0]


def _decoder(h, w1, b1, w2, b2):
    n = h.shape[0]
    dout = w2.shape[1]

    def body(h_r, w1_r, b1_r, w2_r, b2_r, o_r):
        hid = jnp.maximum(h_r[...] @ w1_r[...] + b1_r[...], 0.0)
        o_r[...] = hid @ w2_r[...] + b2_r[...]

    return pl.pallas_call(
        body,
        grid=(n // _BN,),
        in_specs=[
            _row2((_BN, _H)),
            _fix2((_H, _H)), _fix2((1, _H)),
            _fix2((_H, dout)), _fix2((1, dout)),
        ],
        out_specs=_row2((_BN, dout)),
        out_shape=jax.ShapeDtypeStruct((n, dout), jnp.float32),
    )(h, w1, b1.reshape(1, _H), w2, b2.reshape(1, dout))


def _sc_gather_sum(a_tab, b_tab, src, dst):
    e = src.shape[0]
    per_tile = e // _NW
    n_ch = per_tile // _CH
    assert per_tile * _NW == e and n_ch * _CH == per_tile

    mesh = plsc.VectorSubcoreMesh(core_axis_name="c", subcore_axis_name="s")

    @functools.partial(
        pl.kernel,
        mesh=mesh,
        out_type=jax.ShapeDtypeStruct((e, _H), jnp.float32),
        scratch_types=(
            [pltpu.VMEM((_CH,), jnp.int32)] * 4
            + [pltpu.VMEM((_CH, _H), jnp.float32)] * 4
            + [pltpu.SemaphoreType.DMA] * 6
        ),
    )
    def k(a_h, b_h, src_h, dst_h, out_h, *sc):
        idxs = sc[0:2]
        idxd = sc[2:4]
        buf_a = sc[4:6]
        buf_b = sc[6:8]
        sem_a = sc[8:10]
        sem_b = sc[10:12]
        sem_s = sc[12:14]
        c = lax.axis_index("c")
        s = lax.axis_index("s")
        tbase = (c * _NS + s) * per_tile

        def prime(b, i):
            base = pl.multiple_of(tbase + i * _CH, 8)
            pltpu.sync_copy(src_h.at[pl.ds(base, _CH)], idxs[b])
            pltpu.sync_copy(dst_h.at[pl.ds(base, _CH)], idxd[b])
            pltpu.async_copy(a_h.at[idxs[b]], buf_a[b], sem_a[b])
            pltpu.async_copy(b_h.at[idxd[b]], buf_b[b], sem_b[b])

        prime(0, 0)

        def body(b, nb, i):
            base = pl.multiple_of(tbase + i * _CH, 8)

            @pl.when(i >= 1)
            def _():
                pltpu.make_async_copy(buf_a[nb], out_h.at[pl.ds(base, _CH)],
                                      sem_s[nb]).wait()

            prime(nb, jnp.minimum(i + 1, n_ch - 1))
            pltpu.make_async_copy(a_h.at[idxs[b]], buf_a[b], sem_a[b]).wait()
            pltpu.make_async_copy(b_h.at[idxd[b]], buf_b[b], sem_b[b]).wait()

            def add_row(r, _):
                for j in range(_H // 16):
                    sl = pl.ds(j * 16, 16)
                    buf_a[b][r, sl] = buf_a[b][r, sl] + buf_b[b][r, sl]
                return 0

            lax.fori_loop(0, _CH, add_row, 0, unroll=False)
            pltpu.async_copy(buf_a[b], out_h.at[pl.ds(base, _CH)], sem_s[b])

        def outer(g, _):
            body(0, 1, 2 * g)
            body(1, 0, 2 * g + 1)
            return 0

        lax.fori_loop(0, n_ch // 2, outer, 0, unroll=False)
        body(0, 1, n_ch - 1)
        blast, nblast = (n_ch - 1) % 2, 1 - (n_ch - 1) % 2
        base0 = pl.multiple_of(tbase, 8)
        pltpu.make_async_copy(a_h.at[idxs[nblast]], buf_a[nblast],
                              sem_a[nblast]).wait()
        pltpu.make_async_copy(b_h.at[idxd[nblast]], buf_b[nblast],
                              sem_b[nblast]).wait()
        pltpu.make_async_copy(buf_a[blast], out_h.at[pl.ds(base0, _CH)],
                              sem_s[blast]).wait()

    return k(a_tab, b_tab, src, dst)


def _sc_scatter_sum(eu, dst, n_nodes):
    e = eu.shape[0]
    per_tile = e // _NW
    n_ch = per_tile // _CH
    rows_per_sub = -(-n_nodes // (_NS * _ZR)) * _ZR
    n_pad = rows_per_sub * _NS
    n_z = rows_per_sub // _ZR
    assert n_ch * _CH == per_tile and n_z * _ZR == rows_per_sub

    mesh = plsc.VectorSubcoreMesh(core_axis_name="c", subcore_axis_name="s")

    @functools.partial(
        pl.kernel,
        mesh=mesh,
        out_type=jax.ShapeDtypeStruct((_NC, n_pad, _H), jnp.float32),
        scratch_types=(
            [pltpu.VMEM((_CH,), jnp.int32)] * 2
            + [pltpu.VMEM((_CH, _H), jnp.float32)] * 2
            + [pltpu.VMEM((_ZR, _H), jnp.float32),
               pltpu.VMEM_SHARED((n_pad, _H), jnp.float32)]
            + [pltpu.SemaphoreType.DMA] * 4
        ),
    )
    def k(eu_h, dst_h, out_h, *sc):
        idxd = sc[0:2]
        buf = sc[2:4]
        zbuf = sc[4]
        accum = sc[5]
        sem_i = sc[6:8]
        sem_e = sc[8:10]
        c = lax.axis_index("c")
        s = lax.axis_index("s")
        tbase = (c * _NS + s) * per_tile

        def zrow(r, _):
            for j in range(_H // 16):
                zbuf[r, pl.ds(j * 16, 16)] = jnp.zeros((16,), jnp.float32)
            return 0

        lax.fori_loop(0, _ZR, zrow, 0, unroll=False)
        for z in range(n_z):
            pltpu.sync_copy(zbuf, accum.at[pl.ds(s * rows_per_sub + z * _ZR, _ZR)])
        plsc.subcore_barrier()

        def prime(b, i):
            base = pl.multiple_of(tbase + i * _CH, 8)
            pltpu.async_copy(dst_h.at[pl.ds(base, _CH)], idxd[b], sem_i[b])
            pltpu.async_copy(eu_h.at[pl.ds(base, _CH)], buf[b], sem_e[b])

        prime(0, 0)

        def body(b, nb, i):
            prime(nb, jnp.minimum(i + 1, n_ch - 1))
            base = pl.multiple_of(tbase + i * _CH, 8)
            pltpu.make_async_copy(dst_h.at[pl.ds(base, _CH)], idxd[b],
                                  sem_i[b]).wait()
            pltpu.make_async_copy(eu_h.at[pl.ds(base, _CH)], buf[b],
                                  sem_e[b]).wait()
            pltpu.sync_copy(buf[b], accum.at[idxd[b]], add=True)

        def outer(g, _):
            body(0, 1, 2 * g)
            body(1, 0, 2 * g + 1)
            return 0

        lax.fori_loop(0, n_ch // 2, outer, 0, unroll=False)
        body(0, 1, n_ch - 1)
        nblast = 1 - (n_ch - 1) % 2
        base0 = pl.multiple_of(tbase, 8)
        pltpu.make_async_copy(dst_h.at[pl.ds(base0, _CH)], idxd[nblast],
                              sem_i[nblast]).wait()
        pltpu.make_async_copy(eu_h.at[pl.ds(base0, _CH)], buf[nblast],
                              sem_e[nblast]).wait()
        plsc.subcore_barrier()
        pltpu.sync_copy(accum.at[pl.ds(s * rows_per_sub, rows_per_sub)],
                        out_h.at[c, pl.ds(s * rows_per_sub, rows_per_sub)])

    return k(eu, dst)[:, :n_nodes]


def kernel(x, edge_index, edge_attr, params):
    n_nodes = x.shape[0]
    src = edge_index[0]
    dst = edge_index[1]

    pe0 = params["proc0_edge"]
    pn0 = params["proc0_node"]
    pe1 = params["proc1_edge"]
    pn1 = params["proc1_node"]

    h, a0, b0 = _node_enc(x, params["node_enc"], pe0["w1"][:_H], pe0["w1"][_H:2 * _H])
    e = _edge_enc(edge_attr, params["edge_enc"])

    g0 = _sc_gather_sum(a0, b0, src, dst)
    eu0 = _edge_mlp(g0, e, pe0["w1"][2 * _H:], pe0)
    part0 = _sc_scatter_sum(eu0, dst, n_nodes)
    h, a1, b1 = _node_mlp(h, part0[0], part0[1], pn0,
                          pe1["w1"][:_H], pe1["w1"][_H:2 * _H])

    g1 = _sc_gather_sum(a1, b1, src, dst)
    eu1 = _edge_mlp(g1, eu0, pe1["w1"][2 * _H:], pe1)
    part1 = _sc_scatter_sum(eu1, dst, n_nodes)
    h = _node_mlp(h, part1[0], part1[1], pn1)

    return _decoder(h, params["dec_w1"], params["dec_b1"],
                    params["dec_w2"], params["dec_b2"])
